# baseline (speedup 1.0000x reference)
import functools

import jax
import jax.numpy as jnp
from jax import lax
from jax.experimental import pallas as pl
from jax.experimental.pallas import tpu as pltpu
from jax.experimental.pallas import tpu_sc as plsc

SEQ_LEN = 200
HID = 128
TAGS = 1000
ROWS_PER_W = 16
N_CHUNKS = (SEQ_LEN + ROWS_PER_W - 1) // ROWS_PER_W
TAIL = SEQ_LEN - (N_CHUNKS - 1) * ROWS_PER_W


def _sc_gather_partial_sums(sentence, emb_table):
    mesh = plsc.VectorSubcoreMesh(core_axis_name="c", subcore_axis_name="s",
                                  num_cores=1)

    @functools.partial(
        pl.kernel,
        mesh=mesh,
        out_type=jax.ShapeDtypeStruct((N_CHUNKS, HID), jnp.float32),
        scratch_types=[
            pltpu.VMEM((ROWS_PER_W,), jnp.int32),
            pltpu.VMEM((ROWS_PER_W, HID), jnp.float32),
            pltpu.VMEM((HID,), jnp.float32),
            pltpu.SemaphoreType.DMA,
        ],
    )
    def k(sent_hbm, table_hbm, out_hbm, idx_v, rows_v, sum_v, sem):
        wid = lax.axis_index("s")

        def gather_sum(nrows):
            pltpu.sync_copy(sent_hbm.at[pl.ds(wid * ROWS_PER_W, nrows)],
                            idx_v.at[pl.ds(0, nrows)])
            pltpu.async_copy(table_hbm.at[idx_v.at[pl.ds(0, nrows)]],
                             rows_v.at[pl.ds(0, nrows)], sem).wait()
            for d in range(HID // 16):
                acc = rows_v[0, pl.ds(d * 16, 16)]
                for r in range(1, nrows):
                    acc = acc + rows_v[r, pl.ds(d * 16, 16)]
                sum_v[pl.ds(d * 16, 16)] = acc
            pltpu.sync_copy(sum_v, out_hbm.at[wid])

        @pl.when(wid < N_CHUNKS - 1)
        def _():
            gather_sum(ROWS_PER_W)

        @pl.when(wid == N_CHUNKS - 1)
        def _():
            gather_sum(TAIL)

    return k(sentence, emb_table)


def _tc_head(partials, W, b2):
    def body(p_ref, w_ref, b_ref, o_ref):
        vec = jnp.sum(p_ref[...], axis=0, keepdims=True) * (1.0 / SEQ_LEN)
        tag = lax.dot_general(vec, w_ref[...], (((1,), (1,)), ((), ())),
                              preferred_element_type=jnp.float32)
        tag = tag + b_ref[...]
        m = jnp.max(tag, axis=1, keepdims=True)
        e = jnp.exp(tag - m)
        s = jnp.sum(e, axis=1, keepdims=True)
        o_ref[...] = tag - m - jnp.log(s)

    return pl.pallas_call(
        body,
        out_shape=jax.ShapeDtypeStruct((1, TAGS), jnp.float32),
    )(partials, W, b2)


def kernel(sentence, emb_table, W, b):
    sentence = sentence.astype(jnp.int32)
    partials = _sc_gather_partial_sums(sentence, emb_table)
    return _tc_head(partials, W, b.reshape(1, TAGS))
